# baseline (speedup 1.0000x reference)
import functools

import jax
import jax.numpy as jnp
from jax import lax
from jax.experimental import pallas as pl
from jax.experimental.pallas import tpu as pltpu
from jax.experimental.pallas import tpu_sc as plsc

NUM_CORES = 2
NUM_SUBCORES = 16
NUM_WORKERS = NUM_CORES * NUM_SUBCORES

EMBED_DIM = 64
PROJ_DIM = 256
LANES = 16

TC_BLK = 4096
HALF = TC_BLK // 2

IDX_W = 128
GATHERS_PER_ITER = 4
CHUNK = IDX_W * GATHERS_PER_ITER
CHUNKS_PER_BLK = TC_BLK // CHUNK


def _sc_gather(ids1d, table, n_rows):
  per_worker = n_rows // NUM_WORKERS
  iters = per_worker // CHUNK
  blocks_per_worker = per_worker // TC_BLK

  mesh = plsc.VectorSubcoreMesh(core_axis_name="c", subcore_axis_name="s")

  @functools.partial(
      pl.kernel,
      mesh=mesh,
      out_type=jax.ShapeDtypeStruct((n_rows, EMBED_DIM), jnp.float32),
      compiler_params=pltpu.CompilerParams(use_tc_tiling_on_sc=False, needs_layout_passes=False),
      scratch_types=[
          [pltpu.VMEM((CHUNK,), jnp.int32)] * 2,
          [[pltpu.VMEM((IDX_W,), jnp.int32)] * GATHERS_PER_ITER] * 2,
          [pltpu.VMEM((CHUNK, EMBED_DIM), jnp.float32)] * 2,
          [pltpu.SemaphoreType.DMA] * 2,
          [pltpu.SemaphoreType.DMA] * 2,
      ],
  )
  def gather_kernel(ids_hbm, table_hbm, emb_hbm, raw_vs, idx_vss, rows_vs,
                    sem_g, sem_s):
    wid = lax.axis_index("s") * NUM_CORES + lax.axis_index("c")
    blk0 = wid * blocks_per_worker
    row0 = wid * per_worker
    lane2 = 2 * jnp.arange(LANES, dtype=jnp.int32)

    def stage(t, b):
      blk = blk0 + t // CHUNKS_PER_BLK
      sub = t % CHUNKS_PER_BLK
      l_off = blk * TC_BLK + (CHUNK // 2) * sub
      pltpu.sync_copy(ids_hbm.at[pl.ds(l_off, CHUNK // 2)],
                      raw_vs[b].at[pl.ds(0, CHUNK // 2)])
      pltpu.sync_copy(ids_hbm.at[pl.ds(l_off + HALF, CHUNK // 2)],
                      raw_vs[b].at[pl.ds(CHUNK // 2, CHUNK // 2)])
      for v in range(2 * LANES):
        vals = raw_vs[b][pl.ds(LANES * v, LANES)]
        vv = v % LANES
        dst = lane2 + (32 * (vv % 4) + (0 if v < LANES else 1))
        plsc.store_scatter(idx_vss[b][vv // 4], [dst], vals)

    def fire_gathers(b):
      for j in range(GATHERS_PER_ITER):
        pltpu.async_copy(
            table_hbm.at[idx_vss[b][j]],
            rows_vs[b].at[pl.ds(j * IDX_W, IDX_W)],
            sem_g[b])

    def wait_gathers(b):
      for j in range(GATHERS_PER_ITER):
        pltpu.make_async_copy(
            table_hbm.at[idx_vss[b][j]],
            rows_vs[b].at[pl.ds(j * IDX_W, IDX_W)],
            sem_g[b]).wait()

    def out_copy(t, b):
      return pltpu.make_async_copy(
          rows_vs[b], emb_hbm.at[pl.ds(row0 + t * CHUNK, CHUNK)], sem_s[b])

    stage(0, 0)

    def body(k, carry):
      for b in range(2):
        t = 2 * k + b

        @pl.when(t >= 2)
        def _():
          out_copy(t - 2, b).wait()
        fire_gathers(b)

        @pl.when(t + 1 < iters)
        def _():
          stage(t + 1, (b + 1) % 2)
        wait_gathers(b)
        out_copy(t, b).start()
      return carry

    lax.fori_loop(0, iters // 2, body, 0)
    out_copy(iters - 2, 0).wait()
    out_copy(iters - 1, 1).wait()

  return gather_kernel(ids1d, table)


CHUNK_BLOCKS = (1, 2, 4, 6, 6, 6)


def _tc_project_chunk(emb2, wt, prev_out, n_rows, blk0, chunk_rows):
  grid = (chunk_rows // TC_BLK,)

  def matmul_kernel(emb_ref, wt_ref, *refs):
    out_ref = refs[-1]
    blk = emb_ref[...]
    out_ref[0:HALF, :] = jnp.dot(
        blk[:, 0:EMBED_DIM], wt_ref[...], preferred_element_type=jnp.float32)
    out_ref[HALF:TC_BLK, :] = jnp.dot(
        blk[:, EMBED_DIM:2 * EMBED_DIM], wt_ref[...],
        preferred_element_type=jnp.float32)

  in_specs = [
      pl.BlockSpec((HALF, 2 * EMBED_DIM), lambda i: (i, 0)),
      pl.BlockSpec((EMBED_DIM, PROJ_DIM), lambda i: (0, 0)),
  ]
  args = [emb2, wt]
  aliases = {}
  if prev_out is not None:
    in_specs.append(pl.BlockSpec(memory_space=pl.ANY))
    args.append(prev_out)
    aliases = {2: 0}
  return pl.pallas_call(
      matmul_kernel,
      grid=grid,
      in_specs=in_specs,
      out_specs=pl.BlockSpec((TC_BLK, PROJ_DIM), lambda i: (blk0 + i, 0)),
      out_shape=jax.ShapeDtypeStruct((n_rows, PROJ_DIM), jnp.float32),
      input_output_aliases=aliases,
  )(*args)


@jax.jit
def _run(token_ids, embed_table, proj_weight):
  b, l = token_ids.shape
  n = b * l
  ids1d = token_ids.astype(jnp.int32).reshape(n)
  wt = proj_weight.T
  out = None
  row0 = 0
  blk0 = 0
  for ub in CHUNK_BLOCKS:
    chunk_rows = ub * TC_BLK * NUM_WORKERS
    ids_c = lax.slice(ids1d, (row0,), (row0 + chunk_rows,))
    emb = _sc_gather(ids_c, embed_table, chunk_rows)
    emb2 = emb.reshape(chunk_rows // 2, 2 * EMBED_DIM)
    out = _tc_project_chunk(emb2, wt, out, n, blk0, chunk_rows)
    row0 += chunk_rows
    blk0 += chunk_rows // TC_BLK
  return out.reshape(b, l, PROJ_DIM)


def kernel(token_ids, embed_table, proj_weight):
  return _run(token_ids, embed_table, proj_weight)


# speedup vs baseline: 1.0056x; 1.0056x over previous
import functools

import jax
import jax.numpy as jnp
from jax import lax
from jax.experimental import pallas as pl
from jax.experimental.pallas import tpu as pltpu
from jax.experimental.pallas import tpu_sc as plsc

NUM_CORES = 2
NUM_SUBCORES = 16
NUM_WORKERS = NUM_CORES * NUM_SUBCORES

EMBED_DIM = 64
PROJ_DIM = 256
LANES = 16

TC_BLK = 4096
HALF = TC_BLK // 2

IDX_W = 128
GATHERS_PER_ITER = 4
CHUNK = IDX_W * GATHERS_PER_ITER
CHUNKS_PER_BLK = TC_BLK // CHUNK


def _sc_gather(ids1d, table, n_rows):
  per_worker = n_rows // NUM_WORKERS
  iters = per_worker // CHUNK
  blocks_per_worker = per_worker // TC_BLK

  mesh = plsc.VectorSubcoreMesh(core_axis_name="c", subcore_axis_name="s")

  @functools.partial(
      pl.kernel,
      mesh=mesh,
      out_type=jax.ShapeDtypeStruct((n_rows, EMBED_DIM), jnp.float32),
      compiler_params=pltpu.CompilerParams(use_tc_tiling_on_sc=False, needs_layout_passes=False),
      scratch_types=[
          pltpu.VMEM((CHUNK,), jnp.int32),
          [pltpu.VMEM((IDX_W,), jnp.int32)] * GATHERS_PER_ITER,
          pltpu.VMEM((CHUNK, EMBED_DIM), jnp.float32),
          pltpu.SemaphoreType.DMA,
      ],
  )
  def gather_kernel(ids_hbm, table_hbm, emb_hbm, raw_v, idx_vs, rows_v, sem):
    wid = lax.axis_index("s") * NUM_CORES + lax.axis_index("c")
    blk0 = wid * blocks_per_worker
    row0 = wid * per_worker

    def body(t, carry):
      blk = blk0 + t // CHUNKS_PER_BLK
      sub = t % CHUNKS_PER_BLK
      l_off = blk * TC_BLK + (CHUNK // 2) * sub
      pltpu.sync_copy(ids_hbm.at[pl.ds(l_off, CHUNK // 2)],
                      raw_v.at[pl.ds(0, CHUNK // 2)])
      pltpu.sync_copy(ids_hbm.at[pl.ds(l_off + HALF, CHUNK // 2)],
                      raw_v.at[pl.ds(CHUNK // 2, CHUNK // 2)])
      lane2 = 2 * jnp.arange(LANES, dtype=jnp.int32)
      for v in range(2 * LANES):
        vals = raw_v[pl.ds(LANES * v, LANES)]
        vv = v % LANES
        dst = lane2 + (32 * (vv % 4) + (0 if v < LANES else 1))
        plsc.store_scatter(idx_vs[vv // 4], [dst], vals)
      copies = []
      for j in range(GATHERS_PER_ITER):
        copies.append(
            pltpu.async_copy(
                table_hbm.at[idx_vs[j]],
                rows_v.at[pl.ds(j * IDX_W, IDX_W)],
                sem))
      for c in copies:
        c.wait()
      pltpu.sync_copy(rows_v, emb_hbm.at[pl.ds(row0 + t * CHUNK, CHUNK)])
      return carry

    lax.fori_loop(0, iters, body, 0)

  return gather_kernel(ids1d, table)


CHUNK_BLOCKS = (1, 2, 4, 6, 6, 6)


def _tc_project_chunk(emb2, wt, prev_out, n_rows, blk0, chunk_rows):
  grid = (chunk_rows // TC_BLK,)

  def matmul_kernel(emb_ref, wt_ref, *refs):
    out_ref = refs[-1]
    blk = emb_ref[...]
    out_ref[0:HALF, :] = jnp.dot(
        blk[:, 0:EMBED_DIM], wt_ref[...], preferred_element_type=jnp.float32)
    out_ref[HALF:TC_BLK, :] = jnp.dot(
        blk[:, EMBED_DIM:2 * EMBED_DIM], wt_ref[...],
        preferred_element_type=jnp.float32)

  in_specs = [
      pl.BlockSpec((HALF, 2 * EMBED_DIM), lambda i: (i, 0)),
      pl.BlockSpec((EMBED_DIM, PROJ_DIM), lambda i: (0, 0)),
  ]
  args = [emb2, wt]
  aliases = {}
  if prev_out is not None:
    in_specs.append(pl.BlockSpec(memory_space=pl.ANY))
    args.append(prev_out)
    aliases = {2: 0}
  return pl.pallas_call(
      matmul_kernel,
      grid=grid,
      in_specs=in_specs,
      out_specs=pl.BlockSpec((TC_BLK, PROJ_DIM), lambda i: (blk0 + i, 0)),
      out_shape=jax.ShapeDtypeStruct((n_rows, PROJ_DIM), jnp.float32),
      input_output_aliases=aliases,
  )(*args)


@jax.jit
def _run(token_ids, embed_table, proj_weight):
  b, l = token_ids.shape
  n = b * l
  ids1d = token_ids.astype(jnp.int32).reshape(n)
  wt = proj_weight.T
  out = None
  row0 = 0
  blk0 = 0
  for ub in CHUNK_BLOCKS:
    chunk_rows = ub * TC_BLK * NUM_WORKERS
    ids_c = lax.slice(ids1d, (row0,), (row0 + chunk_rows,))
    emb = _sc_gather(ids_c, embed_table, chunk_rows)
    emb2 = emb.reshape(chunk_rows // 2, 2 * EMBED_DIM)
    out = _tc_project_chunk(emb2, wt, out, n, blk0, chunk_rows)
    row0 += chunk_rows
    blk0 += chunk_rows // TC_BLK
  return out.reshape(b, l, PROJ_DIM)


def kernel(token_ids, embed_table, proj_weight):
  return _run(token_ids, embed_table, proj_weight)
